# baseline (speedup 1.0000x reference)
import functools

import jax
import jax.numpy as jnp
from jax.experimental import pallas as pl
from jax.experimental.pallas import tpu as pltpu

_ROW_TILE = 256
_C = 512
_VMEM_LIMIT = 48 * 1024 * 1024


def _bf16_bits(x_f32):
    b = jax.lax.bitcast_convert_type(x_f32, jnp.int32)
    return ((b + 0x7FFF + ((b >> 16) & 1)) >> 16) & 0xFFFF


def _pack_rows(x_f32):
    f = x_f32.shape[1] // 2
    lo = _bf16_bits(x_f32[:, :f])
    hi = _bf16_bits(x_f32[:, f:])
    return lo | (hi << 16)


def _build_plan(edge_index, n, c):
    src, dst = edge_index[0], edge_index[1]
    e = src.shape[0]
    nt = n // _ROW_TILE
    nh = nt // 2
    nc = (e + c - 1) // c + 2 * nh

    key = jnp.where(src != dst, dst, n).astype(jnp.int32)
    sk, ss = jax.lax.sort([key, src.astype(jnp.int32)], num_keys=1)

    bnd = jnp.searchsorted(
        sk, jnp.arange(nt + 1, dtype=jnp.int32) * _ROW_TILE, side="left")
    cnt = bnd[1:] - bnd[:-1]
    ab = (bnd[:-1] >> 7) << 7
    ncpt = jnp.maximum((bnd[1:] - ab + c - 1) // c, 1)
    nc2 = ncpt.reshape(2, nh)
    base2 = jnp.cumsum(nc2, axis=1) - nc2

    car = jnp.arange(nc + 1, dtype=jnp.int32)
    tloc = jax.vmap(
        lambda b: jnp.searchsorted(b, car, side="right"))(base2) - 1
    tloc = jnp.clip(tloc, 0, nh - 1)
    tid = tloc + jnp.arange(2, dtype=jnp.int32)[:, None] * nh

    tg = tid[:, :nc]
    j = car[None, :nc] - jnp.take_along_axis(base2, tloc[:, :nc], axis=1)
    cnt_g = cnt[tg]
    base = ab[tg] + j * c
    lo = jnp.maximum(bnd[tg], base)
    hi = jnp.minimum(bnd[tg] + cnt_g, base + c)
    ne = jnp.clip(hi - lo, 0, c).astype(jnp.int32)

    lpad = (e + 2 * c + 127) // 128 * 128
    sk_p = jnp.concatenate([sk, jnp.full((lpad - e,), n, jnp.int32)])
    ss_p = jnp.concatenate([ss, jnp.zeros((lpad - e,), jnp.int32)])
    rows = ((base >> 7)[..., None]
            + jnp.arange(c // 128, dtype=jnp.int32)).reshape(-1)
    sk_w = jnp.take(sk_p.reshape(-1, 128), rows, axis=0).reshape(2, nc, c)
    ss_w = jnp.take(ss_p.reshape(-1, 128), rows, axis=0).reshape(2, nc, c)
    p = base[..., None] + jnp.arange(c, dtype=jnp.int32)
    valid = (p >= lo[..., None]) & (p < hi[..., None])
    srcp = jnp.where(valid, ss_w, 0).astype(jnp.int32)
    dlp = jnp.where(valid, sk_w - tg[..., None] * _ROW_TILE, -1)

    return (tid.reshape(-1), ne.reshape(-1),
            srcp.reshape(2, nc, 1, c), dlp.reshape(2, nc, 1, c), nc)


def _onehot(dl, w):
    iota = jax.lax.broadcasted_iota(jnp.int32, (_ROW_TILE, w), 0)
    return (iota == dl.reshape(1, w)).astype(jnp.bfloat16)


def _chunk_meta(tid_ref, ne_ref):
    h = pl.program_id(0)
    c = pl.program_id(1)
    nck = pl.num_programs(1)
    base_t = h * (nck + 1)
    tcur = tid_ref[base_t + c]
    is_first = jnp.logical_or(c == 0,
                              tid_ref[base_t + jnp.maximum(c - 1, 0)] != tcur)
    is_last = jnp.logical_or(c == nck - 1, tid_ref[base_t + c + 1] != tcur)
    ne = ne_ref[h * nck + c]
    return h, c, is_first, is_last, ne


def _deg_kernel(tid_ref, ne_ref, dl_ref, o_ref, acc_ref):
    _, _, is_first, is_last, ne = _chunk_meta(tid_ref, ne_ref)

    @pl.when(is_first)
    def _():
        acc_ref[...] = jnp.zeros_like(acc_ref)

    @pl.when(ne > 0)
    def _():
        w = dl_ref.shape[-1]
        ones = jnp.ones((w, 1), jnp.bfloat16)
        acc_ref[...] += jnp.dot(_onehot(dl_ref[...], w), ones,
                                preferred_element_type=jnp.float32)

    @pl.when(is_last)
    def _():
        o_ref[...] = acc_ref[...]


def _gather_accumulate(tid_ref, ne_ref, xw_ref, srcidx_ref, dl_ref,
                       acc_ref, g0, idx_smem, sem, c_edges):
    h, c, is_first, is_last, ne = _chunk_meta(tid_ref, ne_ref)
    nck = pl.num_programs(1)
    buf = jax.lax.rem(c, 3)

    def _issue(cc):
        @pl.when(jnp.logical_and(cc < nck, ne_ref[h * nck + jnp.minimum(
                cc, nck - 1)] > 0))
        def _():
            pltpu.make_async_copy(srcidx_ref.at[h, cc],
                                  idx_smem.at[jax.lax.rem(cc, 3)],
                                  sem.at[jax.lax.rem(cc, 3)]).start()

    @pl.when(c == 0)
    def _():
        _issue(c)
        _issue(c + 1)

    @pl.when(ne > 0)
    def _():
        pltpu.make_async_copy(srcidx_ref.at[h, c], idx_smem.at[buf],
                              sem.at[buf]).wait()

    _issue(c + 2)

    @pl.when(is_first)
    def _():
        acc_ref[...] = jnp.zeros_like(acc_ref)

    @pl.when(ne > 0)
    def _():
        for mi in range(c_edges):
            idx = idx_smem[buf, 0, mi]
            slab = pltpu.bitcast(xw_ref[pl.ds(idx, 1), :], jnp.bfloat16)
            g0[pl.ds(mi, 1), 0:128] = slab[0:1, :]
            g0[pl.ds(mi, 1), 128:256] = slab[1:2, :]
        m = _onehot(dl_ref[...], c_edges)
        acc_ref[...] += jnp.dot(m, g0[...],
                                preferred_element_type=jnp.float32)

    return is_last


def _l1_kernel(tid_ref, ne_ref, xw_ref, srcidx_ref, dl_ref, self_ref,
               dinv_ref, b1_ref, w2_ref, op_ref, ob_ref, acc_ref, g0,
               idx_smem, sem, *, c_edges):
    is_last = _gather_accumulate(tid_ref, ne_ref, xw_ref, srcidx_ref,
                                 dl_ref, acc_ref, g0, idx_smem, sem,
                                 c_edges)

    @pl.when(is_last)
    def _():
        accv = acc_ref[...] + self_ref[...].astype(jnp.float32)
        dv = dinv_ref[...]
        hsig = jnp.maximum(accv * dv + b1_ref[...], 0.0)
        hw2 = jnp.dot(hsig.astype(jnp.bfloat16), w2_ref[...],
                      preferred_element_type=jnp.float32)
        hw2d = hw2 * dv
        op_ref[...] = _pack_rows(hw2d)
        ob_ref[...] = hw2d.astype(jnp.bfloat16)


def _l2_kernel(tid_ref, ne_ref, xw_ref, srcidx_ref, dl_ref, self_ref,
               dinv_ref, b2_ref, o_ref, acc_ref, g0, idx_smem, sem,
               *, c_edges):
    is_last = _gather_accumulate(tid_ref, ne_ref, xw_ref, srcidx_ref,
                                 dl_ref, acc_ref, g0, idx_smem, sem,
                                 c_edges)

    @pl.when(is_last)
    def _():
        accv = acc_ref[...] + self_ref[...].astype(jnp.float32)
        o_ref[...] = accv[:, 0:128] * dinv_ref[...] + b2_ref[...]


def _cparams():
    return pltpu.CompilerParams(
        dimension_semantics=("parallel", "arbitrary"),
        vmem_limit_bytes=_VMEM_LIMIT,
    )


def _tile_spec(width, nc):
    return pl.BlockSpec((_ROW_TILE, width),
                        lambda h, ci, t, e: (t[h * (nc + 1) + ci], 0))


def _agg_call(body, feats_i32, feats_bf16, plan, dinv, extras, extra_specs,
              out_shape, out_specs, c):
    tid, ne, srcp, dlp = plan
    nc = srcp.shape[1]
    w = srcp.shape[-1]

    def _const(shape):
        return pl.BlockSpec(shape, lambda h, ci, t, e: (0,) * len(shape))

    grid_spec = pltpu.PrefetchScalarGridSpec(
        num_scalar_prefetch=2,
        grid=(2, nc),
        in_specs=[
            _const(feats_i32.shape),
            _const(srcp.shape),
            pl.BlockSpec((1, 1, 1, w),
                         lambda h, ci, t, e: (h, ci, 0, 0)),
            _tile_spec(feats_bf16.shape[1], nc),
            _tile_spec(1, nc),
        ] + extra_specs,
        out_specs=out_specs,
        scratch_shapes=[
            pltpu.VMEM((_ROW_TILE, 256), jnp.float32),
            pltpu.VMEM((c, 256), jnp.bfloat16),
            pltpu.SMEM((3, 1, w), jnp.int32),
            pltpu.SemaphoreType.DMA((3,)),
        ],
    )
    return pl.pallas_call(
        functools.partial(body, c_edges=c),
        grid_spec=grid_spec,
        out_shape=out_shape,
        compiler_params=_cparams(),
    )(tid, ne, feats_i32, srcp, dlp, feats_bf16, dinv, *extras)


def _gcn(x, edge_index, w1, b1, w2, b2, c=_C):
    n = x.shape[0]
    tid, ne, srcp, dlp, nc = _build_plan(edge_index, n, c)
    plan = (tid, ne, srcp, dlp)

    deg = pl.pallas_call(
        _deg_kernel,
        grid_spec=pltpu.PrefetchScalarGridSpec(
            num_scalar_prefetch=2,
            grid=(2, nc),
            in_specs=[pl.BlockSpec((1, 1, 1, srcp.shape[-1]),
                                   lambda h, ci, t, e: (h, ci, 0, 0))],
            out_specs=_tile_spec(1, nc),
            scratch_shapes=[pltpu.VMEM((_ROW_TILE, 1), jnp.float32)],
        ),
        out_shape=jax.ShapeDtypeStruct((n, 1), jnp.float32),
        compiler_params=_cparams(),
    )(tid, ne, dlp)
    dinv = jax.lax.rsqrt(deg + 1.0)

    xw1s = dinv * (x @ w1)
    xw_i32 = _pack_rows(xw1s)
    xw_bf = xw1s.astype(jnp.bfloat16)
    b1r = b1.reshape(1, -1).astype(jnp.float32)
    w2p = jnp.pad(w2, ((0, 0), (0, 256 - w2.shape[1]))).astype(jnp.bfloat16)
    b2r = b2.reshape(1, -1).astype(jnp.float32)

    hw2_i32, hw2_bf = _agg_call(
        _l1_kernel, xw_i32, xw_bf, plan, dinv,
        extras=[b1r, w2p],
        extra_specs=[
            pl.BlockSpec((1, 256), lambda h, ci, t, e: (0, 0)),
            pl.BlockSpec((256, 256), lambda h, ci, t, e: (0, 0)),
        ],
        out_shape=(jax.ShapeDtypeStruct((n, 128), jnp.int32),
                   jax.ShapeDtypeStruct((n, 256), jnp.bfloat16)),
        out_specs=(_tile_spec(128, nc), _tile_spec(256, nc)),
        c=c)

    out = _agg_call(
        _l2_kernel, hw2_i32, hw2_bf, plan, dinv,
        extras=[b2r],
        extra_specs=[pl.BlockSpec((1, 128), lambda h, ci, t, e: (0, 0))],
        out_shape=jax.ShapeDtypeStruct((n, 128), jnp.float32),
        out_specs=_tile_spec(128, nc),
        c=c)
    return out


def kernel(x, edge_index, w1, b1, w2, b2):
    return _gcn(x, edge_index, w1, b1, w2, b2)
